# baseline (speedup 1.0000x reference)
import functools

import jax
import jax.numpy as jnp
from jax import lax
from jax.experimental import pallas as pl
from jax.experimental.pallas import tpu as pltpu
from jax.experimental.pallas import tpu_sc as plsc

N_NODES = 10000
D = 128
CHUNK = 128
NC, NS = 2, 16
NW = NC * NS
ACC_ROWS = 10240
SUB_OUT = 632
OUT_ROWS = NS * SUB_OUT
TC_BLK = 1000


def _sc_segment_sum(feature, src_flat, dst_idx, n_chunks):
    mesh = plsc.VectorSubcoreMesh(core_axis_name="c", subcore_axis_name="s")

    @functools.partial(
        pl.kernel,
        mesh=mesh,
        out_type=jax.ShapeDtypeStruct((NC, OUT_ROWS, D), jnp.float32),
        scratch_types=[
            pltpu.VMEM((2, CHUNK), jnp.int32),
            pltpu.VMEM((n_chunks, CHUNK), jnp.int32),
            pltpu.VMEM((CHUNK, D), jnp.float32),
            pltpu.VMEM((CHUNK, D), jnp.float32),
            pltpu.VMEM_SHARED((ACC_ROWS, D), jnp.float32),
            pltpu.SemaphoreType.DMA,
            pltpu.SemaphoreType.DMA,
            pltpu.SemaphoreType.DMA,
            pltpu.SemaphoreType.DMA,
            pltpu.SemaphoreType.DMA,
            pltpu.SemaphoreType.DMA,
        ],
    )
    def k(src_hbm, dst_hbm, feat_hbm, out_hbm, sp, dst_v, rows_a, rows_b,
          acc, semi_a, semi_b, semg_a, semg_b, semsc_a, semsc_b):
        c = lax.axis_index("c")
        s = lax.axis_index("s")
        w = c * NS + s
        base = w * (n_chunks * CHUNK)

        def zrow(i, _):
            for j in range(D // 16):
                rows_a[i, pl.ds(j * 16, 16)] = jnp.zeros((16,), jnp.float32)
            return 0
        lax.fori_loop(0, CHUNK, zrow, 0)

        rows_per_sub = ACC_ROWS // NS
        def zacc(i, _):
            pltpu.sync_copy(
                rows_a, acc.at[pl.ds(s * rows_per_sub + i * CHUNK, CHUNK)])
            return 0
        lax.fori_loop(0, rows_per_sub // CHUNK, zacc, 0)
        plsc.subcore_barrier()

        pltpu.sync_copy(dst_hbm.at[w], dst_v)

        pltpu.async_copy(src_hbm.at[pl.ds(base, CHUNK)], sp.at[0], semi_a)
        pltpu.async_copy(src_hbm.at[pl.ds(base + CHUNK, CHUNK)], sp.at[1],
                         semi_b)
        pltpu.make_async_copy(src_hbm.at[pl.ds(base, CHUNK)], sp.at[0],
                              semi_a).wait()
        pltpu.async_copy(feat_hbm.at[sp.at[0]], rows_a, semg_a)
        pltpu.async_copy(feat_hbm.at[pl.ds(0, CHUNK)], rows_b, semsc_b)

        def half(j, px, py, rows_x, rows_y, semi_x, semi_y, semg_x, semg_y,
                 semsc_x, semsc_y):
            pltpu.make_async_copy(src_hbm.at[pl.ds(base, CHUNK)], sp.at[py],
                                  semi_y).wait()
            pltpu.make_async_copy(rows_y, acc.at[dst_v.at[j]],
                                  semsc_y).wait()
            pltpu.make_async_copy(feat_hbm.at[sp.at[px]], rows_x,
                                  semg_x).wait()
            pltpu.async_copy(rows_x, acc.at[dst_v.at[j]], semsc_x, add=True)
            pltpu.async_copy(feat_hbm.at[sp.at[py]], rows_y, semg_y)
            nxt = jnp.minimum(j + 2, n_chunks - 1)
            pltpu.async_copy(src_hbm.at[pl.ds(base + nxt * CHUNK, CHUNK)],
                             sp.at[px], semi_x)

        def pair(k2, _):
            j = 2 * k2
            half(j, 0, 1, rows_a, rows_b, semi_a, semi_b, semg_a, semg_b,
                 semsc_a, semsc_b)
            half(j + 1, 1, 0, rows_b, rows_a, semi_b, semi_a, semg_b, semg_a,
                 semsc_b, semsc_a)
            return 0
        lax.fori_loop(0, n_chunks // 2, pair, 0)
        pltpu.make_async_copy(rows_b, acc.at[dst_v.at[0]], semsc_b).wait()
        pltpu.make_async_copy(feat_hbm.at[sp.at[0]], rows_a, semg_a).wait()
        pltpu.make_async_copy(src_hbm.at[pl.ds(base, CHUNK)], sp.at[1],
                              semi_b).wait()
        plsc.subcore_barrier()

        pltpu.sync_copy(acc.at[pl.ds(s * SUB_OUT, SUB_OUT)],
                        out_hbm.at[c, pl.ds(s * SUB_OUT, SUB_OUT)])

    return k(src_flat, dst_idx, feature)


def _tc_linear(partials, w_t, b2d):
    def body(p_ref, w_ref, b_ref, o_ref):
        h = p_ref[0] + p_ref[1]
        o_ref[...] = jnp.dot(h, w_ref[...],
                             preferred_element_type=jnp.float32) + b_ref[...]

    return pl.pallas_call(
        body,
        grid=(N_NODES // TC_BLK,),
        in_specs=[
            pl.BlockSpec((NC, TC_BLK, D), lambda i: (0, i, 0)),
            pl.BlockSpec((D, D), lambda i: (0, 0)),
            pl.BlockSpec((1, D), lambda i: (0, 0)),
        ],
        out_specs=pl.BlockSpec((TC_BLK, D), lambda i: (i, 0)),
        out_shape=jax.ShapeDtypeStruct((N_NODES, D), jnp.float32),
    )(partials, w_t, b2d)


def kernel(feature, edge_index, W, b):
    src = edge_index[0].astype(jnp.int32)
    dst = edge_index[1].astype(jnp.int32)
    e = src.shape[0]
    n_chunks = -(-e // (NW * CHUNK))
    n_chunks += n_chunks % 2
    per_w = e // NW
    pad_w = n_chunks * CHUNK - per_w
    pad_dst = N_NODES + (jnp.arange(pad_w, dtype=jnp.int32)
                         % (ACC_ROWS - N_NODES))
    src = jnp.concatenate(
        [src.reshape(NW, per_w),
         jnp.zeros((NW, pad_w), jnp.int32)], axis=1)
    dst = jnp.concatenate(
        [dst.reshape(NW, per_w),
         jnp.broadcast_to(pad_dst, (NW, pad_w))], axis=1)
    src = src.reshape(NW * n_chunks * CHUNK)
    dst = dst.reshape(NW, n_chunks, CHUNK)

    partials = _sc_segment_sum(feature, src, dst, n_chunks)
    return _tc_linear(partials, W.T, b.reshape(1, D))


# speedup vs baseline: 1.2225x; 1.2225x over previous
import functools

import jax
import jax.numpy as jnp
from jax import lax
from jax.experimental import pallas as pl
from jax.experimental.pallas import tpu as pltpu
from jax.experimental.pallas import tpu_sc as plsc

N_NODES = 10000
D = 128
CHUNK = 128
NC, NS = 2, 16
NW = NC * NS
ACC_ROWS = 10240
SUB_OUT = 632
OUT_ROWS = NS * SUB_OUT
TC_BLK = 1000


def _sc_segment_sum(feature, packed_idx, n_chunks):
    mesh = plsc.VectorSubcoreMesh(core_axis_name="c", subcore_axis_name="s")

    @functools.partial(
        pl.kernel,
        mesh=mesh,
        out_type=jax.ShapeDtypeStruct((NC, OUT_ROWS, D), jnp.float32),
        scratch_types=[
            pltpu.VMEM((n_chunks, CHUNK), jnp.int32),
            pltpu.VMEM((2, CHUNK), jnp.int32),
            pltpu.VMEM((2, CHUNK), jnp.int32),
            pltpu.VMEM((CHUNK, D), jnp.float32),
            pltpu.VMEM((CHUNK, D), jnp.float32),
            pltpu.VMEM_SHARED((ACC_ROWS, D), jnp.float32),
            pltpu.SemaphoreType.DMA,
            pltpu.SemaphoreType.DMA,
            pltpu.SemaphoreType.DMA,
            pltpu.SemaphoreType.DMA,
        ],
    )
    def k(packed_hbm, feat_hbm, out_hbm, packed_v, srcb, dstb, rows_a, rows_b,
          acc, semg_a, semg_b, semsc_a, semsc_b):
        c = lax.axis_index("c")
        s = lax.axis_index("s")
        w = c * NS + s

        def zrow(i, _):
            for j in range(D // 16):
                rows_a[i, pl.ds(j * 16, 16)] = jnp.zeros((16,), jnp.float32)
            return 0
        lax.fori_loop(0, CHUNK, zrow, 0)

        rows_per_sub = ACC_ROWS // NS
        def zacc(i, _):
            pltpu.sync_copy(
                rows_a, acc.at[pl.ds(s * rows_per_sub + i * CHUNK, CHUNK)])
            return 0
        lax.fori_loop(0, rows_per_sub // CHUNK, zacc, 0)
        plsc.subcore_barrier()

        pltpu.sync_copy(packed_hbm.at[w], packed_v)

        def unpack(j, p):
            for t in range(CHUNK // 16):
                v = packed_v[j, pl.ds(t * 16, 16)]
                srcb[p, pl.ds(t * 16, 16)] = v & 16383
                dstb[p, pl.ds(t * 16, 16)] = v >> 14

        def chunk(j, _):
            unpack(j, 0)
            pltpu.async_copy(feat_hbm.at[srcb.at[0]], rows_a, semg_a).wait()
            pltpu.sync_copy(rows_a, acc.at[dstb.at[0]], add=True)
            return 0
        lax.fori_loop(0, n_chunks, chunk, 0)
        plsc.subcore_barrier()

        pltpu.sync_copy(acc.at[pl.ds(s * SUB_OUT, SUB_OUT)],
                        out_hbm.at[c, pl.ds(s * SUB_OUT, SUB_OUT)])

    return k(packed_idx, feature)


def _tc_linear(partials, w_t, b2d):
    def body(p_ref, w_ref, b_ref, o_ref):
        h = p_ref[0] + p_ref[1]
        o_ref[...] = jnp.dot(h, w_ref[...],
                             preferred_element_type=jnp.float32) + b_ref[...]

    return pl.pallas_call(
        body,
        grid=(N_NODES // TC_BLK,),
        in_specs=[
            pl.BlockSpec((NC, TC_BLK, D), lambda i: (0, i, 0)),
            pl.BlockSpec((D, D), lambda i: (0, 0)),
            pl.BlockSpec((1, D), lambda i: (0, 0)),
        ],
        out_specs=pl.BlockSpec((TC_BLK, D), lambda i: (i, 0)),
        out_shape=jax.ShapeDtypeStruct((N_NODES, D), jnp.float32),
    )(partials, w_t, b2d)


def kernel(feature, edge_index, W, b):
    src = edge_index[0].astype(jnp.int32)
    dst = edge_index[1].astype(jnp.int32)
    e = src.shape[0]
    n_chunks = -(-e // (NW * CHUNK))
    n_chunks += n_chunks % 2
    per_w = e // NW
    pad_w = n_chunks * CHUNK - per_w
    pad_dst = N_NODES + (jnp.arange(pad_w, dtype=jnp.int32)
                         % (ACC_ROWS - N_NODES))
    src = jnp.concatenate(
        [src.reshape(NW, per_w),
         jnp.zeros((NW, pad_w), jnp.int32)], axis=1)
    dst = jnp.concatenate(
        [dst.reshape(NW, per_w),
         jnp.broadcast_to(pad_dst, (NW, pad_w))], axis=1)
    packed = (src | (dst << 14)).reshape(NW, n_chunks, CHUNK)

    partials = _sc_segment_sum(feature, packed, n_chunks)
    return _tc_linear(partials, W.T, b.reshape(1, D))
